# baseline (speedup 1.0000x reference)
import functools

import jax
import jax.numpy as jnp
from jax import lax
from jax.experimental import pallas as pl
from jax.experimental.pallas import tpu as pltpu
from jax.experimental.pallas import tpu_sc as plsc

_B = 4096
_F = 26
_D = 16
_BF = _B * _F
_NW = 32
_PER_W = _BF // _NW
_NCH = _PER_W // 128
_CV_ROWS = 1000000 // _D
_EPS = 1e-5
_NPAIRS = _F * (_F - 1) // 2

_BB = 512


def _sc_gather_body(idx_hbm, emb_hbm, cvw_hbm, out_e, out_c,
                    idx_v, idx4_v, ebuf, cbuf, sem_e, sem_c):
    wid = lax.axis_index("s") * 2 + lax.axis_index("c")
    pltpu.sync_copy(idx_hbm.at[wid], idx_v)

    @pl.loop(0, _NCH)
    def _shift(j):
        for u in range(8):
            w = idx_v[j, pl.ds(u * 16, 16)]
            idx4_v[j, pl.ds(u * 16, 16)] = lax.shift_right_logical(w, 4)

    @pl.loop(0, 2)
    def _gather(cix):
        hs = []
        for jj in range(13):
            j = cix * 13 + jj
            he = pltpu.async_copy(emb_hbm.at[idx_v.at[j]],
                                  ebuf.at[pl.ds(j * 128, 128)], sem_e)
            hc = pltpu.async_copy(cvw_hbm.at[idx4_v.at[j]],
                                  cbuf.at[pl.ds(j * 128, 128)], sem_c)
            hs.append((he, hc))
        for he, hc in hs:
            he.wait()
            hc.wait()

    base = wid * _PER_W
    pltpu.sync_copy(ebuf, out_e.at[pl.ds(base, _PER_W)])
    pltpu.sync_copy(cbuf, out_c.at[pl.ds(base, _PER_W)])


@functools.lru_cache(maxsize=None)
def _make_sc_gather():
    return pl.kernel(
        _sc_gather_body,
        out_type=[jax.ShapeDtypeStruct((_BF, _D), jnp.float32),
                  jax.ShapeDtypeStruct((_BF, _D), jnp.float32)],
        mesh=plsc.VectorSubcoreMesh(core_axis_name="c", subcore_axis_name="s"),
        compiler_params=pltpu.CompilerParams(use_tc_tiling_on_sc=False),
        scratch_types=[
            pltpu.VMEM((_NCH, 128), jnp.int32),
            pltpu.VMEM((_NCH, 128), jnp.int32),
            pltpu.VMEM((_PER_W, _D), jnp.float32),
            pltpu.VMEM((_PER_W, _D), jnp.float32),
            pltpu.SemaphoreType.DMA,
            pltpu.SemaphoreType.DMA,
        ],
    )


def _tc_body(scal_ref, emb_ref, cblk_ref, feat_ref, out_ref):
    bias = scal_ref[0, 0]
    beta = scal_ref[0, 1]
    c = scal_ref[0, 2]

    T = emb_ref[...].T
    TT = T.reshape(_F, _D, _BB)
    n = jnp.sum(TT * TT, axis=1)
    om = 1.0 - jnp.clip(n, 0.0, 1.0 - _EPS)

    acc = jnp.zeros((_F, _BB), jnp.float32)
    for s in range(1, _F // 2 + 1):
        rolled = jnp.concatenate([T[_D * s:], T[:_D * s]], axis=0)
        omr = jnp.concatenate([om[s:], om[:s]], axis=0)
        RT = rolled.reshape(_F, _D, _BB)
        d = TT - RT
        sqd = jnp.sum(d * d, axis=1)
        x = sqd / (om * omr) * 2.0 + 1.0
        z = jnp.sqrt(jnp.maximum(x * x - 1.0, 0.0))
        dist = jnp.log(x + z)
        acc = acc + (dist if s < _F // 2 else 0.5 * dist)
    pair_sum = jnp.sum(acc, axis=0)

    colT = (feat_ref[...] & 15).astype(jnp.float32).T
    CT = cblk_ref[...].T.reshape(_F, _D, _BB)
    dmat = lax.broadcasted_iota(jnp.int32, (_F, _D, _BB), 1).astype(jnp.float32)
    picked = jnp.where(dmat == colT[:, None, :], CT, 0.0)
    lin = jnp.sum(jnp.sum(picked, axis=1), axis=0)

    out_ref[...] = beta * pair_sum + lin + (c * _NPAIRS + bias)


def _tc_compute(scal, emb_g, cblk, feats):
    return pl.pallas_call(
        _tc_body,
        grid=(_B // _BB,),
        in_specs=[
            pl.BlockSpec(memory_space=pltpu.SMEM),
            pl.BlockSpec((_BB, _F * _D), lambda i: (i, 0)),
            pl.BlockSpec((_BB, _F * _D), lambda i: (i, 0)),
            pl.BlockSpec((_BB, _F), lambda i: (i, 0)),
        ],
        out_specs=pl.BlockSpec((_BB,), lambda i: (i,)),
        out_shape=jax.ShapeDtypeStruct((_B,), jnp.float32),
    )(scal, emb_g, cblk, feats)


def kernel(features, emb_table, coeff_table, bias, beta, c):
    feats = features.astype(jnp.int32)
    idx3 = feats.reshape(_NW, _NCH, 128)
    cview = coeff_table.reshape(_CV_ROWS, _D)
    emb_g, cblk = _make_sc_gather()(idx3, emb_table, cview)
    scal = jnp.concatenate([bias, beta, c]).reshape(1, 3).astype(jnp.float32)
    return _tc_compute(scal, emb_g.reshape(_B, _F * _D),
                       cblk.reshape(_B, _F * _D), feats)
